# Initial kernel scaffold; baseline (speedup 1.0000x reference)
import jax
import jax.numpy as jnp
from jax.experimental import pallas as pl


def kernel(logits, gt, threshold, temperature_single):
    raise NotImplementedError("write your pallas kernel here")


# speedup vs baseline: 1.0093x; 1.0093x over previous
import functools

import jax
import jax.numpy as jnp
from jax import lax
from jax.experimental import pallas as pl
from jax.experimental.pallas import tpu as pltpu
from jax.experimental.pallas import tpu_sc as plsc

N = 131072
C = 150
CP = 256
RB = 1024
ROWS_2D = N // 128

NC = 2
NS = 16
NW = NC * NS
RPW = N // NW
G = 128
CHUNKS = RPW // G


def _entropy_body(params_ref, x_ref, g_ref, pre_ref, cond_ref):
    x = x_ref[...]
    thr = params_ref[0, 0]
    invt = params_ref[0, 1]
    m = jnp.max(x, axis=1, keepdims=True)
    e = jnp.exp(x - m)
    s = jnp.sum(e, axis=1, keepdims=True)
    t = jnp.sum(e * (x - m), axis=1, keepdims=True)
    ent = jnp.log(s) - t / s
    cond = ent < thr
    row = jnp.where(cond, x * invt, jnp.float32(1.0))
    gbits = lax.bitcast_convert_type(g_ref[...], jnp.float32)
    pad = jnp.zeros((RB, CP - C - 1), jnp.float32)
    pre_ref[...] = jnp.concatenate([row, gbits, pad], axis=1)
    cond_ref[...] = cond.astype(jnp.int32)


def _dest_body(cond_ref, dest_ref):
    c = cond_ref[...].astype(jnp.float32)
    r, l = ROWS_2D, 128
    u = (lax.broadcasted_iota(jnp.int32, (l, l), 0)
         <= lax.broadcasted_iota(jnp.int32, (l, l), 1)).astype(jnp.float32)
    cs = lax.dot_general(c, u, (((1,), (0,)), ((), ())))
    rs = jnp.sum(c, axis=1, keepdims=True)
    lo = (lax.broadcasted_iota(jnp.int32, (r, r), 0)
          > lax.broadcasted_iota(jnp.int32, (r, r), 1)).astype(jnp.float32)
    ro = lax.dot_general(lo, rs, (((1,), (0,)), ((), ())))
    cc = cs + ro
    k = jnp.max(cc)
    v = (lax.broadcasted_iota(jnp.int32, (r, l), 0) * l
         + lax.broadcasted_iota(jnp.int32, (r, l), 1)).astype(jnp.float32)
    dest = jnp.where(c > 0.5, cc - 1.0, k + v - cc)
    dest_ref[...] = dest.astype(jnp.int32)


def _sc_scatter_body(pre_hbm, dest_hbm, out_hbm, rows_v, idx_v, sem):
    wid = lax.axis_index("s") * NC + lax.axis_index("c")
    base0 = wid * RPW

    def chunk(i, carry):
        base = base0 + i * G
        pltpu.sync_copy(dest_hbm.at[pl.ds(base, G)], idx_v)
        pltpu.sync_copy(pre_hbm.at[pl.ds(base, G)], rows_v)
        pltpu.async_copy(rows_v, out_hbm.at[idx_v], sem).wait()
        return carry

    lax.fori_loop(0, CHUNKS, chunk, 0)


@functools.cache
def _sc_scatter():
    return pl.kernel(
        _sc_scatter_body,
        out_type=jax.ShapeDtypeStruct((N, CP), jnp.float32),
        mesh=plsc.VectorSubcoreMesh(core_axis_name="c", subcore_axis_name="s"),
        scratch_types=[
            pltpu.VMEM((G, CP), jnp.float32),
            pltpu.VMEM((G,), jnp.int32),
            pltpu.SemaphoreType.DMA,
        ],
    )


def kernel(logits, gt, threshold, temperature_single):
    x2 = jnp.transpose(logits, (0, 2, 3, 1)).reshape(N, C)
    y2 = gt.reshape(N, 1)
    thr = jnp.asarray(threshold, jnp.float32)
    invt = jnp.float32(1.0) / temperature_single[0].astype(jnp.float32)
    params = jnp.stack([thr, invt]).reshape(1, 2)

    pre, cond = pl.pallas_call(
        _entropy_body,
        grid=(N // RB,),
        in_specs=[
            pl.BlockSpec(memory_space=pltpu.SMEM),
            pl.BlockSpec((RB, C), lambda i: (i, 0)),
            pl.BlockSpec((RB, 1), lambda i: (i, 0)),
        ],
        out_specs=[
            pl.BlockSpec((RB, CP), lambda i: (i, 0)),
            pl.BlockSpec((RB, 1), lambda i: (i, 0)),
        ],
        out_shape=[
            jax.ShapeDtypeStruct((N, CP), jnp.float32),
            jax.ShapeDtypeStruct((N, 1), jnp.int32),
        ],
    )(params, x2, y2)

    dest2d = pl.pallas_call(
        _dest_body,
        in_specs=[pl.BlockSpec((ROWS_2D, 128), lambda: (0, 0))],
        out_specs=pl.BlockSpec((ROWS_2D, 128), lambda: (0, 0)),
        out_shape=jax.ShapeDtypeStruct((ROWS_2D, 128), jnp.int32),
    )(cond.reshape(ROWS_2D, 128))

    out_pad = _sc_scatter()(pre, dest2d.reshape(N))
    cal_logits = out_pad[:, :C]
    cal_gt = lax.bitcast_convert_type(out_pad[:, C], jnp.int32)
    return (cal_logits, cal_gt)
